# baseline (speedup 1.0000x reference)
import jax
import jax.numpy as jnp
from jax import lax
from jax.experimental import pallas as pl
from jax.experimental.pallas import tpu as pltpu
from jax.experimental.pallas import tpu_sc as plsc

_B = 64
_N = 100
_D = 256
_NC = 2
_NS = 16
_ROWS_PER_W = _B // (_NC * _NS)


def _sc_body(emb_hbm, target_hbm, pe_hbm, emb_v, zero_v, sem_in, sem_out):
    wid = lax.axis_index("s") * _NC + lax.axis_index("c")
    in_copy = pltpu.async_copy(emb_hbm, emb_v, sem_in)

    z = jnp.zeros((16,), jnp.float32)

    def zero_row(r, carry):
        def zero_chunk(c, carry2):
            zero_v[r, pl.ds(c * 16, 16)] = z
            return carry2

        return lax.fori_loop(0, _D // 16, zero_chunk, carry)

    lax.fori_loop(0, _N, zero_row, 0)

    b0 = wid * _ROWS_PER_W
    t0 = pltpu.async_copy(zero_v, target_hbm.at[b0], sem_out)
    t1 = pltpu.async_copy(zero_v, target_hbm.at[b0 + 1], sem_out)
    in_copy.wait()
    p0 = pltpu.async_copy(emb_v, pe_hbm.at[b0], sem_out)
    p1 = pltpu.async_copy(emb_v, pe_hbm.at[b0 + 1], sem_out)
    t0.wait()
    t1.wait()
    p0.wait()
    p1.wait()


def kernel(x, emb_table):
    out = jax.ShapeDtypeStruct((_B, _N, _D), jnp.float32)
    mesh = plsc.VectorSubcoreMesh(core_axis_name="c", subcore_axis_name="s")
    sc_call = pl.kernel(
        _sc_body,
        mesh=mesh,
        out_type=[out, out],
        scratch_types=[
            pltpu.VMEM((_N, _D), jnp.float32),
            pltpu.VMEM((_N, _D), jnp.float32),
            pltpu.SemaphoreType.DMA,
            pltpu.SemaphoreType.DMA,
        ],
    )
    target, target_pe = sc_call(emb_table)
    return (target, target_pe)


# speedup vs baseline: 1.0802x; 1.0802x over previous
import jax
import jax.numpy as jnp
from jax import lax
from jax.experimental import pallas as pl
from jax.experimental.pallas import tpu as pltpu
from jax.experimental.pallas import tpu_sc as plsc

_B = 64
_N = 100
_D = 256
_NC = 2
_NS = 16
_ROWS_PER_W = _B // (_NC * _NS)


def _sc_pe_body(emb_hbm, pe_hbm, emb_v, sem_in, sem_out):
    wid = lax.axis_index("s") * _NC + lax.axis_index("c")
    pltpu.async_copy(emb_hbm, emb_v, sem_in).wait()
    b0 = wid * _ROWS_PER_W
    p0 = pltpu.async_copy(emb_v, pe_hbm.at[b0], sem_out)
    p1 = pltpu.async_copy(emb_v, pe_hbm.at[b0 + 1], sem_out)
    p0.wait()
    p1.wait()


def _tc_zeros_body(target_ref):
    target_ref[...] = jnp.zeros(target_ref.shape, target_ref.dtype)


def kernel(x, emb_table):
    out = jax.ShapeDtypeStruct((_B, _N, _D), jnp.float32)

    mesh = plsc.VectorSubcoreMesh(core_axis_name="c", subcore_axis_name="s")
    sc_call = pl.kernel(
        _sc_pe_body,
        mesh=mesh,
        out_type=out,
        scratch_types=[
            pltpu.VMEM((_N, _D), jnp.float32),
            pltpu.SemaphoreType.DMA,
            pltpu.SemaphoreType.DMA,
        ],
    )
    target_pe = sc_call(emb_table)

    bb = 16
    target = pl.pallas_call(
        _tc_zeros_body,
        grid=(_B // bb,),
        out_specs=pl.BlockSpec((bb, _N, _D), lambda i: (i, 0, 0)),
        out_shape=out,
    )()
    return (target, target_pe)


# speedup vs baseline: 5.2625x; 4.8720x over previous
import jax
import jax.numpy as jnp
from jax.experimental import pallas as pl

_B = 64
_N = 100
_D = 256


def _tc_body(emb_ref, zt_ref, pet_ref):
    zt_ref[...] = jnp.zeros(zt_ref.shape, zt_ref.dtype)
    pet_ref[...] = jnp.broadcast_to(emb_ref[...], pet_ref.shape)


def kernel(x, emb_table):
    nb = 20
    out_t = jax.ShapeDtypeStruct((_N, _B, _D), jnp.float32)
    zt, pet = pl.pallas_call(
        _tc_body,
        grid=(_N // nb,),
        in_specs=[pl.BlockSpec((nb, 1, _D), lambda i: (i, 0, 0))],
        out_specs=[
            pl.BlockSpec((nb, _B, _D), lambda i: (i, 0, 0)),
            pl.BlockSpec((nb, _B, _D), lambda i: (i, 0, 0)),
        ],
        out_shape=[out_t, out_t],
    )(emb_table.reshape(_N, 1, _D))
    return (jnp.transpose(zt, (1, 0, 2)), jnp.transpose(pet, (1, 0, 2)))
